# Initial kernel scaffold; baseline (speedup 1.0000x reference)
import jax
import jax.numpy as jnp
from jax.experimental import pallas as pl


def kernel(x, edge_index, Ws1, Wn1, b1, Ws2, Wn2, b2, Ws3, Wn3, b3):
    raise NotImplementedError("write your pallas kernel here")


# speedup vs baseline: 4.6370x; 4.6370x over previous
import functools

import jax
import jax.numpy as jnp
from jax import lax
from jax.experimental import pallas as pl
from jax.experimental.pallas import tpu as pltpu
from jax.experimental.pallas import tpu_sc as plsc

_N = 10000
_E = 320000
_NC = 2
_NS = 16
_NW = _NC * _NS
_EPW = _E // _NW
_CH = 128
_NFULL = _EPW // _CH
_TAIL = _EPW - _NFULL * _CH
_RPT = 624
_REM = _N - _NS * _RPT


def _make_sc_agg(d):
  mesh = plsc.VectorSubcoreMesh(core_axis_name="c", subcore_axis_name="s")
  out_type = [jax.ShapeDtypeStruct((_NC, _N, d), jnp.float32)]
  scratch = [
      pltpu.VMEM_SHARED((_N, d), jnp.float32),
      pltpu.VMEM((_CH,), jnp.int32),
      pltpu.VMEM((_CH,), jnp.int32),
      pltpu.VMEM((_TAIL,), jnp.int32),
      pltpu.VMEM((_TAIL,), jnp.int32),
      pltpu.VMEM((_CH, d), jnp.float32),
      pltpu.VMEM((_TAIL, d), jnp.float32),
      pltpu.SemaphoreType.DMA,
  ]
  def body(*refs):
    (h, src, dst, zrows, out_acc,
     acc_sh, src_v, dst_v, srcT, dstT, rows_v, rowsT, sem) = refs

    c = lax.axis_index("c")
    s = lax.axis_index("s")
    wid = s * _NC + c
    base = wid * _EPW
    row0 = s * _RPT

    pltpu.sync_copy(zrows, acc_sh.at[pl.ds(row0, _RPT)])

    @pl.when(s == _NS - 1)
    def _():
      pltpu.sync_copy(zrows.at[pl.ds(0, _REM)],
                      acc_sh.at[pl.ds(_NS * _RPT, _REM)])

    plsc.subcore_barrier()

    def chunk(g, carry):
      off = base + g * _CH
      pltpu.sync_copy(src.at[pl.ds(off, _CH)], src_v)
      pltpu.sync_copy(dst.at[pl.ds(off, _CH)], dst_v)
      pltpu.async_copy(h.at[src_v], rows_v, sem).wait()
      pltpu.sync_copy(rows_v, acc_sh.at[dst_v], add=True)
      return carry

    lax.fori_loop(0, _NFULL, chunk, 0)

    offT = base + _NFULL * _CH
    pltpu.sync_copy(src.at[pl.ds(offT, _TAIL)], srcT)
    pltpu.sync_copy(dst.at[pl.ds(offT, _TAIL)], dstT)
    pltpu.async_copy(h.at[srcT], rowsT, sem).wait()
    pltpu.sync_copy(rowsT, acc_sh.at[dstT], add=True)
    plsc.subcore_barrier()

    pltpu.sync_copy(acc_sh.at[pl.ds(row0, _RPT)],
                    out_acc.at[c].at[pl.ds(row0, _RPT)])

    @pl.when(s == _NS - 1)
    def _():
      pltpu.sync_copy(acc_sh.at[pl.ds(_NS * _RPT, _REM)],
                      out_acc.at[c].at[pl.ds(_NS * _RPT, _REM)])

  k = pl.kernel(body, out_type=out_type, mesh=mesh, scratch_types=scratch,
                compiler_params=pltpu.CompilerParams(needs_layout_passes=False))
  return lambda *a: k(*a)[0]


_sc_agg128 = _make_sc_agg(128)

_BN = 1024


def _mean(acc_ref, deg_ref):
  acc = acc_ref[0] + acc_ref[1]
  d = deg_ref[0] + deg_ref[1]
  deg = d[:, 0]
  inv = 1.0 / jnp.maximum(deg, 1.0)
  return acc * inv[:, None]


def _tc1_body(x_ref, acc_ref, deg_ref, ws_ref, wn_ref, b_ref, o_ref):
  agg = _mean(acc_ref, deg_ref)
  h = (jnp.dot(x_ref[...], ws_ref[...], preferred_element_type=jnp.float32)
       + jnp.dot(agg, wn_ref[...], preferred_element_type=jnp.float32)
       + b_ref[...])
  o_ref[...] = jnp.maximum(h, 0.0)


def _tc3_body(x_ref, acc_ref, deg_ref, ws_ref, wn_ref, b_ref, o_ref):
  agg = _mean(acc_ref, deg_ref)
  z = (jnp.dot(x_ref[...], ws_ref[...], preferred_element_type=jnp.float32)
       + jnp.dot(agg, wn_ref[...], preferred_element_type=jnp.float32)
       + b_ref[...])
  col = lax.broadcasted_iota(jnp.int32, z.shape, 1)
  valid = col < 47
  zm = jnp.where(valid, z, -1e30)
  m = jnp.max(zm, axis=1, keepdims=True)
  e = jnp.where(valid, jnp.exp(zm - m), 0.0)
  lse = jnp.log(jnp.sum(e, axis=1, keepdims=True)) + m
  o_ref[...] = z - lse


def _row_spec(d):
  return pl.BlockSpec((_BN, d), lambda i: (i, 0))


def _full_spec(r, c):
  return pl.BlockSpec((r, c), lambda i: (0, 0))


def _acc_spec(d):
  return pl.BlockSpec((_NC, _BN, d), lambda i: (0, i, 0))


_GRID = (pl.cdiv(_N, _BN),)

_tc1 = pl.pallas_call(
    _tc1_body,
    grid=_GRID,
    in_specs=[_row_spec(128), _acc_spec(128), _acc_spec(128),
              _full_spec(128, 128), _full_spec(128, 128), _full_spec(1, 128)],
    out_specs=_row_spec(128),
    out_shape=jax.ShapeDtypeStruct((_N, 128), jnp.float32),
)

_tc3 = pl.pallas_call(
    _tc3_body,
    grid=_GRID,
    in_specs=[_row_spec(128), _acc_spec(128), _acc_spec(128),
              _full_spec(128, 128), _full_spec(128, 128), _full_spec(1, 128)],
    out_specs=_row_spec(128),
    out_shape=jax.ShapeDtypeStruct((_N, 128), jnp.float32),
)


def kernel(x, edge_index, Ws1, Wn1, b1, Ws2, Wn2, b2, Ws3, Wn3, b3):
  src = edge_index[0].astype(jnp.int32)
  dst = edge_index[1].astype(jnp.int32)
  z128 = jnp.zeros((_RPT, 128), jnp.float32)
  ones_tab = jnp.ones((_N, 128), jnp.float32)

  degacc = _sc_agg128(ones_tab, src, dst, z128)
  accx = _sc_agg128(x, src, dst, z128)
  h1 = _tc1(x, accx, degacc, Ws1, Wn1, b1.reshape(1, -1))

  acch = _sc_agg128(h1, src, dst, z128)
  h2 = _tc1(h1, acch, degacc, Ws2, Wn2, b2.reshape(1, -1))

  acc2 = _sc_agg128(h2, src, dst, z128)
  ws3p = jnp.zeros((128, 128), jnp.float32).at[:, :47].set(Ws3)
  wn3p = jnp.zeros((128, 128), jnp.float32).at[:, :47].set(Wn3)
  b3p = jnp.zeros((1, 128), jnp.float32).at[0, :47].set(b3)
  z = _tc3(h2, acc2, degacc, ws3p, wn3p, b3p)
  return z[:, :47]
